# Initial kernel scaffold; baseline (speedup 1.0000x reference)
import jax
import jax.numpy as jnp
from jax.experimental import pallas as pl


def kernel(hidden_states, w_router, gate_w, up_w, down_w, shared_gate_w, shared_up_w, shared_down_w):
    raise NotImplementedError("write your pallas kernel here")


# speedup vs baseline: 1.8705x; 1.8705x over previous
import functools

import jax
import jax.numpy as jnp
from jax.experimental import pallas as pl
from jax.experimental.pallas import tpu as pltpu

D = 1024
E = 8
F = 512
FS = 1024
SCALE = 2.5
BM = 1024


def _moe_body(x_ref, wr_ref, gw_ref, uw_ref, dw_ref, sg_ref, su_ref, sd_ref,
              out_ref, comb_ref, acc_ref):
    e = pl.program_id(1)
    x32 = x_ref[...]
    xb = x32.astype(jnp.bfloat16)

    @pl.when(e == 0)
    def _():
        logits = jnp.dot(x32, wr_ref[...], preferred_element_type=jnp.float32)
        m = jnp.max(logits, axis=-1, keepdims=True)
        p = jnp.exp(logits - m)
        scores = p / jnp.sum(p, axis=-1, keepdims=True)
        lane = jax.lax.broadcasted_iota(jnp.int32, scores.shape, 1)
        v1 = jnp.max(scores, axis=-1, keepdims=True)
        i1 = jnp.min(jnp.where(scores == v1, lane, E), axis=-1, keepdims=True)
        masked = jnp.where(lane == i1, -1e30, scores)
        v2 = jnp.max(masked, axis=-1, keepdims=True)
        i2 = jnp.min(jnp.where(masked == v2, lane, E), axis=-1, keepdims=True)
        s = v1 + v2
        comb = (jnp.where(lane == i1, v1 / s, 0.0)
                + jnp.where(lane == i2, v2 / s, 0.0))
        comb_ref[...] = comb * SCALE
        g = jnp.dot(xb, sg_ref[...], preferred_element_type=jnp.float32)
        u = jnp.dot(xb, su_ref[...], preferred_element_type=jnp.float32)
        h = (g * jax.nn.sigmoid(g) * u).astype(jnp.bfloat16)
        acc_ref[...] = jnp.dot(h, sd_ref[...], preferred_element_type=jnp.float32)

    g = jnp.dot(xb, gw_ref[0], preferred_element_type=jnp.float32)
    u = jnp.dot(xb, uw_ref[0], preferred_element_type=jnp.float32)
    h = (g * jax.nn.sigmoid(g) * u).astype(jnp.bfloat16)
    r = jnp.dot(h, dw_ref[0], preferred_element_type=jnp.float32)
    eidx = jax.lax.broadcasted_iota(jnp.int32, (BM, E), 1)
    factor = jnp.sum(jnp.where(eidx == e, comb_ref[...], 0.0), axis=-1,
                     keepdims=True)
    acc_ref[...] += r * factor

    @pl.when(e == E - 1)
    def _():
        out_ref[...] = acc_ref[...]


def kernel(hidden_states, w_router, gate_w, up_w, down_w,
           shared_gate_w, shared_up_w, shared_down_w):
    shape = hidden_states.shape
    x2 = hidden_states.reshape(-1, D)
    n = x2.shape[0]
    rb = n // BM
    gw = gate_w.astype(jnp.bfloat16)
    uw = up_w.astype(jnp.bfloat16)
    dw = down_w.astype(jnp.bfloat16)
    sg = shared_gate_w.astype(jnp.bfloat16)
    su = shared_up_w.astype(jnp.bfloat16)
    sd = shared_down_w.astype(jnp.bfloat16)

    out = pl.pallas_call(
        _moe_body,
        grid=(rb, E),
        in_specs=[
            pl.BlockSpec((BM, D), lambda i, e: (i, 0)),
            pl.BlockSpec((D, E), lambda i, e: (0, 0)),
            pl.BlockSpec((1, D, F), lambda i, e: (e, 0, 0)),
            pl.BlockSpec((1, D, F), lambda i, e: (e, 0, 0)),
            pl.BlockSpec((1, F, D), lambda i, e: (e, 0, 0)),
            pl.BlockSpec((D, FS), lambda i, e: (0, 0)),
            pl.BlockSpec((D, FS), lambda i, e: (0, 0)),
            pl.BlockSpec((FS, D), lambda i, e: (0, 0)),
        ],
        out_specs=pl.BlockSpec((BM, D), lambda i, e: (i, 0)),
        out_shape=jax.ShapeDtypeStruct((n, D), jnp.float32),
        scratch_shapes=[
            pltpu.VMEM((BM, E), jnp.float32),
            pltpu.VMEM((BM, D), jnp.float32),
        ],
        compiler_params=pltpu.CompilerParams(
            dimension_semantics=("parallel", "arbitrary"),
        ),
    )(x2, w_router, gw, uw, dw, sg, su, sd)
    return out.reshape(shape)
